# baseline (speedup 1.0000x reference)
import jax
import jax.numpy as jnp
from jax import lax
from jax.experimental import pallas as pl
from jax.experimental.pallas import tpu as pltpu
from jax.experimental.pallas import tpu_sc as plsc

_SDF = 128
_NPTS = 16
_NC = 2
_NS = 16
_NW = _NC * _NS
_BAND = 16
_NBANDS = _SDF // _BAND


def _sc_body(xs_hbm, ys_hbm, r0_hbm, r1_hbm, o0_hbm, o1_hbm, ztile_hbm,
             out_hbm, xs_v, ys_v, r0_v, r1_v, o0_v, o1_v, buf0, buf1,
             sem0, sem1):
    bpw = 256 // _NW
    wid = lax.axis_index("s") * _NC + lax.axis_index("c")
    base_b = wid * bpw
    pltpu.sync_copy(xs_hbm.at[pl.ds(base_b, bpw)], xs_v)
    pltpu.sync_copy(ys_hbm.at[pl.ds(base_b, bpw)], ys_v)
    pltpu.sync_copy(r0_hbm.at[pl.ds(base_b, bpw)], r0_v)
    pltpu.sync_copy(r1_hbm.at[pl.ds(base_b, bpw)], r1_v)
    pltpu.sync_copy(o0_hbm.at[pl.ds(base_b, bpw)], o0_v)
    pltpu.sync_copy(o1_hbm.at[pl.ds(base_b, bpw)], o1_v)
    pltpu.sync_copy(ztile_hbm, buf0)
    pltpu.sync_copy(ztile_hbm, buf1)
    bufs = (buf0, buf1)
    sems = (sem0, sem1)
    ones = jnp.ones((_NPTS,), jnp.float32)
    zeros = jnp.zeros((_NPTS,), jnp.float32)
    p = lax.iota(jnp.int32, _NPTS)
    pending = [None, None]
    slot = 0
    for k in range(bpw):
        xk = xs_v[k]
        yk = ys_v[k]
        row = jnp.clip((yk / r0_v[k] + o0_v[k]).astype(jnp.int32), 0, _SDF - 1)
        col = jnp.clip((xk / r1_v[k] + o1_v[k]).astype(jnp.int32), 0, _SDF - 1)
        for band in range(_NBANDS):
            lo = band * _BAND
            m = (row >= lo) & (row < lo + _BAND)
            lr = jnp.clip(row - lo, 0, _BAND - 1)
            if pending[slot] is not None:
                h, (plr, pcol, pm) = pending[slot]
                h.wait()
                plsc.store_scatter(bufs[slot], [plr, p, pcol], zeros, mask=pm)
            plsc.store_scatter(bufs[slot], [lr, p, col], ones, mask=m)
            h = pltpu.async_copy(
                bufs[slot], out_hbm.at[base_b + k, pl.ds(lo, _BAND)],
                sems[slot])
            pending[slot] = (h, (lr, col, m))
            slot ^= 1
    for s in range(2):
        if pending[s] is not None:
            pending[s][0].wait()


def kernel(x, resolution, origin):
    b = x.shape[0]
    pts = x.reshape(b, _NPTS, 2)
    xs = pts[:, :, 0]
    ys = pts[:, :, 1]
    rs0 = jnp.broadcast_to(resolution[:, 0:1], (b, _NPTS))
    rs1 = jnp.broadcast_to(resolution[:, 1:2], (b, _NPTS))
    og0 = jnp.broadcast_to(origin[:, 0:1], (b, _NPTS))
    og1 = jnp.broadcast_to(origin[:, 1:2], (b, _NPTS))
    ztile = jnp.zeros((_BAND, _NPTS, _SDF), jnp.float32)
    mesh = plsc.VectorSubcoreMesh(
        core_axis_name="c", subcore_axis_name="s",
        num_cores=_NC, num_subcores=_NS)
    out = pl.kernel(
        _sc_body,
        out_type=jax.ShapeDtypeStruct((b, _SDF, _NPTS, _SDF), jnp.float32),
        mesh=mesh,
        compiler_params=pltpu.CompilerParams(needs_layout_passes=False),
        scratch_types=[
            pltpu.VMEM((b // _NW, _NPTS), jnp.float32),
            pltpu.VMEM((b // _NW, _NPTS), jnp.float32),
            pltpu.VMEM((b // _NW, _NPTS), jnp.float32),
            pltpu.VMEM((b // _NW, _NPTS), jnp.float32),
            pltpu.VMEM((b // _NW, _NPTS), jnp.float32),
            pltpu.VMEM((b // _NW, _NPTS), jnp.float32),
            pltpu.VMEM((_BAND, _NPTS, _SDF), jnp.float32),
            pltpu.VMEM((_BAND, _NPTS, _SDF), jnp.float32),
            pltpu.SemaphoreType.DMA,
            pltpu.SemaphoreType.DMA,
        ],
    )(xs, ys, rs0, rs1, og0, og1, ztile)
    return jnp.transpose(out, (0, 1, 3, 2))


# speedup vs baseline: 1.0941x; 1.0941x over previous
import jax
import jax.numpy as jnp
from jax import lax
from jax.experimental import pallas as pl
from jax.experimental.pallas import tpu as pltpu
from jax.experimental.pallas import tpu_sc as plsc

_SDF = 128
_NPTS = 16
_NC = 2
_NS = 16
_NW = _NC * _NS
_BAND = 32


def _sc_body(xs_hbm, ys_hbm, r0_hbm, r1_hbm, o0_hbm, o1_hbm, ztile_hbm,
             out_hbm, xs_v, ys_v, r0_v, r1_v, o0_v, o1_v, buf):
    bpw = 256 // _NW
    wid = lax.axis_index("s") * _NC + lax.axis_index("c")
    base_b = wid * bpw
    pltpu.sync_copy(xs_hbm.at[pl.ds(base_b, bpw)], xs_v)
    pltpu.sync_copy(ys_hbm.at[pl.ds(base_b, bpw)], ys_v)
    pltpu.sync_copy(r0_hbm.at[pl.ds(base_b, bpw)], r0_v)
    pltpu.sync_copy(r1_hbm.at[pl.ds(base_b, bpw)], r1_v)
    pltpu.sync_copy(o0_hbm.at[pl.ds(base_b, bpw)], o0_v)
    pltpu.sync_copy(o1_hbm.at[pl.ds(base_b, bpw)], o1_v)
    pltpu.sync_copy(ztile_hbm, buf)
    ones = jnp.ones((_NPTS,), jnp.float32)
    zeros = jnp.zeros((_NPTS,), jnp.float32)
    p = lax.iota(jnp.int32, _NPTS)
    for k in range(bpw):
        xk = xs_v[k]
        yk = ys_v[k]
        row = jnp.clip((yk / r0_v[k] + o0_v[k]).astype(jnp.int32), 0, _SDF - 1)
        col = jnp.clip((xk / r1_v[k] + o1_v[k]).astype(jnp.int32), 0, _SDF - 1)
        for band in range(_SDF // _BAND):
            lo = band * _BAND
            m = (row >= lo) & (row < lo + _BAND)
            lr = jnp.clip(row - lo, 0, _BAND - 1)
            plsc.store_scatter(buf, [lr, p, col], ones, mask=m)
            pltpu.sync_copy(buf, out_hbm.at[base_b + k, pl.ds(lo, _BAND)])
            plsc.store_scatter(buf, [lr, p, col], zeros, mask=m)


def kernel(x, resolution, origin):
    b = x.shape[0]
    pts = x.reshape(b, _NPTS, 2)
    xs = pts[:, :, 0]
    ys = pts[:, :, 1]
    rs0 = jnp.broadcast_to(resolution[:, 0:1], (b, _NPTS))
    rs1 = jnp.broadcast_to(resolution[:, 1:2], (b, _NPTS))
    og0 = jnp.broadcast_to(origin[:, 0:1], (b, _NPTS))
    og1 = jnp.broadcast_to(origin[:, 1:2], (b, _NPTS))
    ztile = jnp.zeros((_BAND, _NPTS, _SDF), jnp.float32)
    mesh = plsc.VectorSubcoreMesh(
        core_axis_name="c", subcore_axis_name="s",
        num_cores=_NC, num_subcores=_NS)
    out = pl.kernel(
        _sc_body,
        out_type=jax.ShapeDtypeStruct((b, _SDF, _NPTS, _SDF), jnp.float32),
        mesh=mesh,
        compiler_params=pltpu.CompilerParams(needs_layout_passes=False),
        scratch_types=[
            pltpu.VMEM((b // _NW, _NPTS), jnp.float32),
            pltpu.VMEM((b // _NW, _NPTS), jnp.float32),
            pltpu.VMEM((b // _NW, _NPTS), jnp.float32),
            pltpu.VMEM((b // _NW, _NPTS), jnp.float32),
            pltpu.VMEM((b // _NW, _NPTS), jnp.float32),
            pltpu.VMEM((b // _NW, _NPTS), jnp.float32),
            pltpu.VMEM((_BAND, _NPTS, _SDF), jnp.float32),
        ],
    )(xs, ys, rs0, rs1, og0, og1, ztile)
    return jnp.transpose(out, (0, 1, 3, 2))
